# baseline (speedup 1.0000x reference)
import jax
import jax.numpy as jnp
from jax import lax
from jax.experimental import pallas as pl
from jax.experimental.pallas import tpu as pltpu
from jax.experimental.pallas import tpu_sc as plsc

NUM_ROWS = 1000001
DIM = 32
BATCH = 4096
N_P = 200
B_TOTAL = BATCH * N_P
NC, NS = 2, 16
NW = NC * NS
BLOCK = 1280
N_BLOCKS = B_TOTAL // (NW * BLOCK)
B_PER_W = N_BLOCKS * BLOCK
NBUF = 2
N_GROUPS = N_BLOCKS // NBUF

ROWS_PAD = 1000064
WCOLS = 1664
WGRID = ROWS_PAD // WCOLS
PBLK = 4


def _sc_body(idx_hbm, table_hbm, out_hbm, idx_v, *scratch):
    bufs = scratch[:NBUF]
    sems = scratch[NBUF:]
    wid = lax.axis_index("s") * NC + lax.axis_index("c")
    base = wid * B_PER_W
    pltpu.sync_copy(idx_hbm.at[wid], idx_v)

    for b in range(NBUF):
        pltpu.make_async_copy(table_hbm.at[idx_v.at[b]], bufs[b], sems[b]).start()

    def group(g):
        k0 = g * NBUF
        for b in range(NBUF):
            k = k0 + b
            pltpu.make_async_copy(
                table_hbm.at[pl.ds(0, BLOCK)], bufs[b], sems[b]
            ).wait()
            pltpu.sync_copy(bufs[b], out_hbm.at[pl.ds(base + k * BLOCK, BLOCK)])
            nxt = k + NBUF

            @pl.when(nxt < N_BLOCKS)
            def _():
                pltpu.make_async_copy(
                    table_hbm.at[idx_v.at[nxt]], bufs[b], sems[b]
                ).start()

    pl.loop(0, N_GROUPS)(group)


def _ot_body(x_ref, o_ref):
    o_ref[...] = x_ref[...].T.reshape(PBLK, DIM, BATCH)


def _wt_body(w_ref, eye_ref, o_ref):
    o_ref[...] = jax.lax.dot_general(
        w_ref[...],
        eye_ref[...],
        (((0,), (0,)), ((), ())),
        preferred_element_type=jnp.float32,
    )


@jax.jit
def _lookup(indices_blocked, weight_t):
    eye = jnp.eye(DIM, dtype=jnp.float32)
    table = pl.pallas_call(
        _wt_body,
        grid=(WGRID,),
        in_specs=[
            pl.BlockSpec((DIM, WCOLS), lambda j: (0, j)),
            pl.BlockSpec((DIM, DIM), lambda j: (0, 0)),
        ],
        out_specs=pl.BlockSpec((WCOLS, DIM), lambda j: (j, 0)),
        out_shape=jax.ShapeDtypeStruct((ROWS_PAD, DIM), jnp.float32),
    )(weight_t, eye)

    mesh = plsc.VectorSubcoreMesh(core_axis_name="c", subcore_axis_name="s")
    flat = pl.kernel(
        _sc_body,
        out_type=jax.ShapeDtypeStruct((B_TOTAL, DIM), jnp.float32),
        mesh=mesh,
        scratch_types=[pltpu.VMEM((N_BLOCKS, BLOCK), jnp.int32)]
        + [pltpu.VMEM((BLOCK, DIM), jnp.float32) for _ in range(NBUF)]
        + [pltpu.SemaphoreType.DMA for _ in range(NBUF)],
        compiler_params=pltpu.CompilerParams(use_tc_tiling_on_sc=False),
    )(indices_blocked, table)

    x2 = flat.reshape(BATCH, N_P * DIM)
    out_t = pl.pallas_call(
        _ot_body,
        grid=(N_P // PBLK,),
        in_specs=[pl.BlockSpec((BATCH, PBLK * DIM), lambda p: (0, p))],
        out_specs=pl.BlockSpec((PBLK, DIM, BATCH), lambda p: (p, 0, 0)),
        out_shape=jax.ShapeDtypeStruct((N_P, DIM, BATCH), jnp.float32),
    )(x2)
    return out_t


def kernel(indices, weight):
    idx_blocked = indices.reshape(NW, N_BLOCKS, BLOCK)
    out_t = _lookup(idx_blocked, weight.T)
    return jnp.transpose(out_t, (2, 0, 1))


# speedup vs baseline: 1.4866x; 1.4866x over previous
import jax
import jax.numpy as jnp
from jax import lax
from jax.experimental import pallas as pl
from jax.experimental.pallas import tpu as pltpu
from jax.experimental.pallas import tpu_sc as plsc

NUM_ROWS = 1000001
DIM = 32
BATCH = 4096
N_P = 200
B_TOTAL = BATCH * N_P
NC, NS = 2, 16
NW = NC * NS
BLOCK = 1280
N_BLOCKS = B_TOTAL // (NW * BLOCK)
B_PER_W = N_BLOCKS * BLOCK
NBUF = 2
N_GROUPS = N_BLOCKS // NBUF

PBLK = 4


def _sc_body(idx_hbm, table_hbm, out_hbm, idx_v, *scratch):
    bufs = scratch[:NBUF]
    sems = scratch[NBUF:]
    wid = lax.axis_index("s") * NC + lax.axis_index("c")
    base = wid * B_PER_W
    pltpu.sync_copy(idx_hbm.at[wid], idx_v)

    for b in range(NBUF):
        pltpu.make_async_copy(table_hbm.at[idx_v.at[b]], bufs[b], sems[b]).start()

    def group(g):
        k0 = g * NBUF
        for b in range(NBUF):
            k = k0 + b
            pltpu.make_async_copy(
                table_hbm.at[pl.ds(0, BLOCK)], bufs[b], sems[b]
            ).wait()
            pltpu.sync_copy(bufs[b], out_hbm.at[pl.ds(base + k * BLOCK, BLOCK)])
            nxt = k + NBUF

            @pl.when(nxt < N_BLOCKS)
            def _():
                pltpu.make_async_copy(
                    table_hbm.at[idx_v.at[nxt]], bufs[b], sems[b]
                ).start()

    pl.loop(0, N_GROUPS)(group)


def _ot_body(x_ref, o_ref):
    o_ref[...] = x_ref[...].T.reshape(PBLK, DIM, BATCH)


@jax.jit
def _lookup(indices_blocked, table):
    mesh = plsc.VectorSubcoreMesh(core_axis_name="c", subcore_axis_name="s")
    flat = pl.kernel(
        _sc_body,
        out_type=jax.ShapeDtypeStruct((B_TOTAL, DIM), jnp.float32),
        mesh=mesh,
        scratch_types=[pltpu.VMEM((N_BLOCKS, BLOCK), jnp.int32)]
        + [pltpu.VMEM((BLOCK, DIM), jnp.float32) for _ in range(NBUF)]
        + [pltpu.SemaphoreType.DMA for _ in range(NBUF)],
        compiler_params=pltpu.CompilerParams(use_tc_tiling_on_sc=False),
    )(indices_blocked, table)

    x2 = flat.reshape(BATCH, N_P * DIM)
    out_t = pl.pallas_call(
        _ot_body,
        grid=(N_P // PBLK,),
        in_specs=[pl.BlockSpec((BATCH, PBLK * DIM), lambda p: (0, p))],
        out_specs=pl.BlockSpec((PBLK, DIM, BATCH), lambda p: (p, 0, 0)),
        out_shape=jax.ShapeDtypeStruct((N_P, DIM, BATCH), jnp.float32),
    )(x2)
    return out_t


def kernel(indices, weight):
    idx_blocked = indices.reshape(NW, N_BLOCKS, BLOCK)
    out_t = _lookup(idx_blocked, weight)
    return jnp.transpose(out_t, (2, 0, 1))


# speedup vs baseline: 1.5093x; 1.0152x over previous
import jax
import jax.numpy as jnp
from jax import lax
from jax.experimental import pallas as pl
from jax.experimental.pallas import tpu as pltpu
from jax.experimental.pallas import tpu_sc as plsc

NUM_ROWS = 1000001
DIM = 32
BATCH = 4096
N_P = 200
B_TOTAL = BATCH * N_P
NC, NS = 2, 16
NW = NC * NS
BLOCK = 1280
N_BLOCKS = B_TOTAL // (NW * BLOCK)
B_PER_W = N_BLOCKS * BLOCK
NBUF = 2
N_GROUPS = N_BLOCKS // NBUF

PBLK = 8


def _sc_body(idx_hbm, table_hbm, out_hbm, idx_v, *scratch):
    bufs = scratch[:NBUF]
    sems = scratch[NBUF:]
    wid = lax.axis_index("s") * NC + lax.axis_index("c")
    base = wid * B_PER_W
    pltpu.sync_copy(idx_hbm.at[wid], idx_v)

    for b in range(NBUF):
        pltpu.make_async_copy(table_hbm.at[idx_v.at[b]], bufs[b], sems[b]).start()

    def group(g):
        k0 = g * NBUF
        for b in range(NBUF):
            k = k0 + b
            pltpu.make_async_copy(
                table_hbm.at[pl.ds(0, BLOCK)], bufs[b], sems[b]
            ).wait()
            pltpu.sync_copy(bufs[b], out_hbm.at[pl.ds(base + k * BLOCK, BLOCK)])
            nxt = k + NBUF

            @pl.when(nxt < N_BLOCKS)
            def _():
                pltpu.make_async_copy(
                    table_hbm.at[idx_v.at[nxt]], bufs[b], sems[b]
                ).start()

    pl.loop(0, N_GROUPS)(group)


def _ot_body(x_ref, o_ref):
    o_ref[...] = x_ref[...].T.reshape(PBLK, DIM, BATCH)


@jax.jit
def _lookup(indices_blocked, table):
    mesh = plsc.VectorSubcoreMesh(core_axis_name="c", subcore_axis_name="s")
    flat = pl.kernel(
        _sc_body,
        out_type=jax.ShapeDtypeStruct((B_TOTAL, DIM), jnp.float32),
        mesh=mesh,
        scratch_types=[pltpu.VMEM((N_BLOCKS, BLOCK), jnp.int32)]
        + [pltpu.VMEM((BLOCK, DIM), jnp.float32) for _ in range(NBUF)]
        + [pltpu.SemaphoreType.DMA for _ in range(NBUF)],
        compiler_params=pltpu.CompilerParams(use_tc_tiling_on_sc=False),
    )(indices_blocked, table)

    x2 = flat.reshape(BATCH, N_P * DIM)
    out_t = pl.pallas_call(
        _ot_body,
        grid=(N_P // PBLK,),
        in_specs=[pl.BlockSpec((BATCH, PBLK * DIM), lambda p: (0, p))],
        out_specs=pl.BlockSpec((PBLK, DIM, BATCH), lambda p: (p, 0, 0)),
        out_shape=jax.ShapeDtypeStruct((N_P, DIM, BATCH), jnp.float32),
    )(x2)
    return out_t


def kernel(indices, weight):
    idx_blocked = indices.reshape(NW, N_BLOCKS, BLOCK)
    out_t = _lookup(idx_blocked, weight)
    return jnp.transpose(out_t, (2, 0, 1))


# speedup vs baseline: 1.5140x; 1.0031x over previous
import jax
import jax.numpy as jnp
from jax import lax
from jax.experimental import pallas as pl
from jax.experimental.pallas import tpu as pltpu
from jax.experimental.pallas import tpu_sc as plsc

NUM_ROWS = 1000001
DIM = 32
BATCH = 4096
N_P = 200
B_TOTAL = BATCH * N_P
NC, NS = 2, 16
NW = NC * NS
BLOCK = 1280
N_BLOCKS = B_TOTAL // (NW * BLOCK)
B_PER_W = N_BLOCKS * BLOCK
NBUF = 2
N_GROUPS = N_BLOCKS // NBUF

PBLK = 20


def _sc_body(idx_hbm, table_hbm, out_hbm, idx_v, *scratch):
    bufs = scratch[:NBUF]
    sems = scratch[NBUF:]
    wid = lax.axis_index("s") * NC + lax.axis_index("c")
    base = wid * B_PER_W
    pltpu.sync_copy(idx_hbm.at[wid], idx_v)

    for b in range(NBUF):
        pltpu.make_async_copy(table_hbm.at[idx_v.at[b]], bufs[b], sems[b]).start()

    def group(g):
        k0 = g * NBUF
        for b in range(NBUF):
            k = k0 + b
            pltpu.make_async_copy(
                table_hbm.at[pl.ds(0, BLOCK)], bufs[b], sems[b]
            ).wait()
            pltpu.sync_copy(bufs[b], out_hbm.at[pl.ds(base + k * BLOCK, BLOCK)])
            nxt = k + NBUF

            @pl.when(nxt < N_BLOCKS)
            def _():
                pltpu.make_async_copy(
                    table_hbm.at[idx_v.at[nxt]], bufs[b], sems[b]
                ).start()

    pl.loop(0, N_GROUPS)(group)


def _ot_body(x_ref, o_ref):
    o_ref[...] = x_ref[...].T.reshape(PBLK, DIM, BATCH)


@jax.jit
def _lookup(indices_blocked, table):
    mesh = plsc.VectorSubcoreMesh(core_axis_name="c", subcore_axis_name="s")
    flat = pl.kernel(
        _sc_body,
        out_type=jax.ShapeDtypeStruct((B_TOTAL, DIM), jnp.float32),
        mesh=mesh,
        scratch_types=[pltpu.VMEM((N_BLOCKS, BLOCK), jnp.int32)]
        + [pltpu.VMEM((BLOCK, DIM), jnp.float32) for _ in range(NBUF)]
        + [pltpu.SemaphoreType.DMA for _ in range(NBUF)],
        compiler_params=pltpu.CompilerParams(use_tc_tiling_on_sc=False),
    )(indices_blocked, table)

    x2 = flat.reshape(BATCH, N_P * DIM)
    out_t = pl.pallas_call(
        _ot_body,
        grid=(N_P // PBLK,),
        in_specs=[pl.BlockSpec((BATCH, PBLK * DIM), lambda p: (0, p))],
        out_specs=pl.BlockSpec((PBLK, DIM, BATCH), lambda p: (p, 0, 0)),
        out_shape=jax.ShapeDtypeStruct((N_P, DIM, BATCH), jnp.float32),
    )(x2)
    return out_t


def kernel(indices, weight):
    idx_blocked = indices.reshape(NW, N_BLOCKS, BLOCK)
    out_t = _lookup(idx_blocked, weight)
    return jnp.transpose(out_t, (2, 0, 1))
